# baseline (speedup 1.0000x reference)
import functools

import jax
import jax.numpy as jnp
from jax import lax
from jax.experimental import pallas as pl
from jax.experimental.pallas import tpu as pltpu
from jax.experimental.pallas import tpu_sc as plsc

N = 2048
DIM = 128
MDIM = 16
K = 3
PWM = 20
EIN = 2 * DIM + 1
H1 = 2 * EIN
BR = 256
NBLK = N // BR


def _silu(x):
    return x * (0.5 * jnp.tanh(0.5 * x) + 0.5)


def _knn_body(cpad_ref, idx_ref, dsel_ref, ct_s):
    b = pl.program_id(0)

    @pl.when(b == 0)
    def _():
        ct_s[...] = cpad_ref[...].T

    cr = cpad_ref[pl.ds(b * BR, BR), :]
    ct = ct_s[...]
    acc = jnp.zeros((BR, N), jnp.float32)
    for d in range(3):
        diff = cr[:, d:d + 1] - ct[d:d + 1, :]
        acc = acc + diff * diff
    col = lax.broadcasted_iota(jnp.int32, (BR, N), 1)
    dist = acc
    idxs = []
    vals = []
    for t in range(K):
        m = jnp.min(dist, axis=1, keepdims=True)
        am = jnp.argmin(dist, axis=1, keepdims=True).astype(jnp.int32)
        vals.append(m)
        idxs.append(am)
        if t + 1 < K:
            dist = jnp.where(col == am, jnp.inf, dist)
    idx_ref[...] = jnp.concatenate(idxs, axis=1).T
    dsel_ref[...] = jnp.concatenate(vals, axis=1).T


def _knn(cpad):
    return pl.pallas_call(
        _knn_body,
        grid=(NBLK,),
        in_specs=[
            pl.BlockSpec((N, 8), lambda b: (0, 0)),
        ],
        out_specs=[
            pl.BlockSpec((K, BR), lambda b: (0, b)),
            pl.BlockSpec((K, BR), lambda b: (0, b)),
        ],
        out_shape=[
            jax.ShapeDtypeStruct((K, N), jnp.int32),
            jax.ShapeDtypeStruct((K, N), jnp.float32),
        ],
        scratch_shapes=[pltpu.VMEM((8, N), jnp.float32)],
    )(cpad)


def _make_sc_gather():
    info = plsc.get_sparse_core_info()
    nc, ns = info.num_cores, info.num_subcores
    nw = nc * ns
    b_total = K * N
    b_per_w = b_total // nw
    mesh = plsc.VectorSubcoreMesh(core_axis_name="c", subcore_axis_name="s")

    @functools.partial(
        pl.kernel,
        mesh=mesh,
        out_type=jax.ShapeDtypeStruct((b_total, DIM), jnp.float32),
        scratch_types=[
            pltpu.VMEM((b_per_w,), jnp.int32),
            pltpu.VMEM((b_per_w, DIM), jnp.float32),
            pltpu.SemaphoreType.DMA,
        ],
    )
    def gather(table_hbm, idx_hbm, out_hbm, idx_v, rows_v, sem):
        wid = lax.axis_index("s") * nc + lax.axis_index("c")
        base = wid * b_per_w
        pltpu.sync_copy(idx_hbm.at[pl.ds(base, b_per_w)], idx_v)
        pltpu.async_copy(table_hbm.at[idx_v], rows_v, sem).wait()
        pltpu.sync_copy(rows_v, out_hbm.at[pl.ds(base, b_per_w)])

    return gather


_SC_GATHER_CACHE = []


def _gather_sc(table, idx):
    if not _SC_GATHER_CACHE:
        _SC_GATHER_CACHE.append(_make_sc_gather())
    return _SC_GATHER_CACHE[0](table, idx)


def _layer_compute(f_ref, fj_ref, dsel_ref, ew1_ref, eb1_ref, ew2_ref,
                   eb2_ref, nw1_ref, nb1_ref, nw2_ref, nb2_ref):
    f = f_ref[...]
    wa = ew1_ref[0:DIM]
    wb = ew1_ref[DIM:2 * DIM]
    wd = ew1_ref[2 * DIM:2 * DIM + 1]
    ua = jnp.dot(f, wa, preferred_element_type=jnp.float32)
    fj_all = jnp.concatenate([fj_ref[0], fj_ref[1], fj_ref[2]],
                             axis=0)
    dk = jnp.concatenate([dsel_ref[0], dsel_ref[1], dsel_ref[2]],
                         axis=0)
    z = (jnp.concatenate([ua, ua, ua], axis=0)
         + jnp.dot(fj_all, wb, preferred_element_type=jnp.float32)
         + dk * wd
         + eb1_ref[...][None, :])
    m = _silu(z)
    mk = _silu(jnp.dot(m, ew2_ref[...], preferred_element_type=jnp.float32)
               + eb2_ref[...][None, :])
    mi = mk[0:BR] + mk[BR:2 * BR] + mk[2 * BR:3 * BR]
    h = _silu(jnp.dot(f, nw1_ref[0:DIM], preferred_element_type=jnp.float32)
              + jnp.dot(mi, nw1_ref[DIM:DIM + MDIM],
                        preferred_element_type=jnp.float32)
              + nb1_ref[...][None, :])
    return f + jnp.dot(h, nw2_ref[...],
                       preferred_element_type=jnp.float32) \
        + nb2_ref[...][None, :]


def _layer_body(f_ref, fj_ref, dsel_ref, ew1_ref, eb1_ref, ew2_ref, eb2_ref,
                nw1_ref, nb1_ref, nw2_ref, nb2_ref, out_ref):
    out_ref[...] = _layer_compute(f_ref, fj_ref, dsel_ref, ew1_ref, eb1_ref,
                                  ew2_ref, eb2_ref, nw1_ref, nb1_ref,
                                  nw2_ref, nb2_ref)


def _layer_final_body(f_ref, fj_ref, dsel_ref, ew1_ref, eb1_ref, ew2_ref,
                      eb2_ref, nw1_ref, nb1_ref, nw2_ref, nb2_ref,
                      lw_ref, lb_ref, out_ref):
    fnew = _layer_compute(f_ref, fj_ref, dsel_ref, ew1_ref, eb1_ref,
                          ew2_ref, eb2_ref, nw1_ref, nb1_ref,
                          nw2_ref, nb2_ref)
    out_ref[...] = jnp.dot(fnew, lw_ref[...],
                           preferred_element_type=jnp.float32) \
        + lb_ref[...][None, :]


def _layer_specs():
    def full(shape):
        nzero = len(shape)
        return pl.BlockSpec(shape, lambda b, _n=nzero: (0,) * _n)
    return [
        pl.BlockSpec((BR, DIM), lambda b: (b, 0)),
        pl.BlockSpec((K, BR, DIM), lambda b: (0, b, 0)),
        pl.BlockSpec((K, BR, 1), lambda b: (0, b, 0)),
        full((EIN, H1)),
        full((H1,)),
        full((H1, MDIM)),
        full((MDIM,)),
        full((DIM + MDIM, 2 * DIM)),
        full((2 * DIM,)),
        full((2 * DIM, DIM)),
        full((DIM,)),
    ]


def _layer(f, fj, dsel3, p):
    return pl.pallas_call(
        _layer_body,
        grid=(NBLK,),
        in_specs=_layer_specs(),
        out_specs=pl.BlockSpec((BR, DIM), lambda b: (b, 0)),
        out_shape=jax.ShapeDtypeStruct((N, DIM), jnp.float32),
    )(f, fj, dsel3, p['eW1'], p['eb1'], p['eW2'], p['eb2'],
      p['nW1'], p['nb1'], p['nW2'], p['nb2'])


def _layer_final(f, fj, dsel3, p, lw, lb):
    def full(shape):
        nzero = len(shape)
        return pl.BlockSpec(shape, lambda b, _n=nzero: (0,) * _n)
    return pl.pallas_call(
        _layer_final_body,
        grid=(NBLK,),
        in_specs=_layer_specs() + [full((DIM, PWM)), full((PWM,))],
        out_specs=pl.BlockSpec((BR, PWM), lambda b: (b, 0)),
        out_shape=jax.ShapeDtypeStruct((N, PWM), jnp.float32),
    )(f, fj, dsel3, p['eW1'], p['eb1'], p['eW2'], p['eb2'],
      p['nW1'], p['nb1'], p['nW2'], p['nb2'], lw, lb)


def kernel(feats, coords, params):
    f = feats[0]
    c = coords[0]
    cpad = jnp.pad(c, ((0, 0), (0, 5)))

    idx, dsel = _knn(cpad)
    idx_flat = idx.reshape(-1)
    dsel3 = dsel[:, :, None]

    layers = params['layers']
    for p in layers[:-1]:
        fj = _gather_sc(f, idx_flat).reshape(K, N, DIM)
        f = _layer(f, fj, dsel3, p)
    fj = _gather_sc(f, idx_flat).reshape(K, N, DIM)
    out = _layer_final(f, fj, dsel3, layers[-1],
                       params['lin']['W'], params['lin']['b'])
    return out[None]


# speedup vs baseline: 1.2515x; 1.2515x over previous
import functools

import jax
import jax.numpy as jnp
from jax import lax
from jax.experimental import pallas as pl
from jax.experimental.pallas import tpu as pltpu
from jax.experimental.pallas import tpu_sc as plsc

N = 2048
DIM = 128
MDIM = 16
K = 3
PWM = 20
EIN = 2 * DIM + 1
H1 = 2 * EIN
BR = 256
NBLK = N // BR
BRL = 1024
NBLKL = N // BRL


def _silu(x):
    return x * (0.5 * jnp.tanh(0.5 * x) + 0.5)


def _knn_body(cpad_ref, ct_ref, idx_ref, dsel_ref):
    cr = cpad_ref[...]
    ct = ct_ref[...]
    acc = jnp.zeros((BR, N), jnp.float32)
    for d in range(3):
        diff = cr[:, d:d + 1] - ct[d:d + 1, :]
        acc = acc + diff * diff
    col = lax.broadcasted_iota(jnp.int32, (BR, N), 1)
    dist = acc
    idxs = []
    vals = []
    for t in range(K):
        m = jnp.min(dist, axis=1, keepdims=True)
        am = jnp.min(jnp.where(dist == m, col, N), axis=1,
                     keepdims=True)
        vals.append(m)
        idxs.append(am)
        if t + 1 < K:
            dist = jnp.where(col == am, jnp.inf, dist)
    idx_ref[...] = jnp.concatenate(idxs, axis=1).T
    dsel_ref[...] = jnp.concatenate(vals, axis=1).T


def _knn(cpad, ct):
    return pl.pallas_call(
        _knn_body,
        grid=(NBLK,),
        in_specs=[
            pl.BlockSpec((BR, 8), lambda b: (b, 0)),
            pl.BlockSpec((8, N), lambda b: (0, 0)),
        ],
        out_specs=[
            pl.BlockSpec((K, BR), lambda b: (0, b)),
            pl.BlockSpec((K, BR), lambda b: (0, b)),
        ],
        out_shape=[
            jax.ShapeDtypeStruct((K, N), jnp.int32),
            jax.ShapeDtypeStruct((K, N), jnp.float32),
        ],
    )(cpad, ct)


def _make_sc_gather():
    info = plsc.get_sparse_core_info()
    nc, ns = info.num_cores, info.num_subcores
    nw = nc * ns
    b_total = K * N
    b_per_w = b_total // nw
    mesh = plsc.VectorSubcoreMesh(core_axis_name="c", subcore_axis_name="s")

    @functools.partial(
        pl.kernel,
        mesh=mesh,
        out_type=jax.ShapeDtypeStruct((b_total, DIM), jnp.float32),
        scratch_types=[
            pltpu.VMEM((b_per_w,), jnp.int32),
            pltpu.VMEM((b_per_w, DIM), jnp.float32),
            pltpu.SemaphoreType.DMA,
        ],
    )
    def gather(table_hbm, idx_hbm, out_hbm, idx_v, rows_v, sem):
        wid = lax.axis_index("s") * nc + lax.axis_index("c")
        base = wid * b_per_w
        pltpu.sync_copy(idx_hbm.at[pl.ds(base, b_per_w)], idx_v)
        pltpu.async_copy(table_hbm.at[idx_v], rows_v, sem).wait()
        pltpu.sync_copy(rows_v, out_hbm.at[pl.ds(base, b_per_w)])

    return gather


_SC_GATHER_CACHE = []


def _gather_sc(table, idx):
    if not _SC_GATHER_CACHE:
        _SC_GATHER_CACHE.append(_make_sc_gather())
    return _SC_GATHER_CACHE[0](table, idx)


def _layer_compute(f_ref, fj_ref, dsel_ref, ew1_ref, eb1_ref, ew2_ref,
                   eb2_ref, nw1_ref, nb1_ref, nw2_ref, nb2_ref):
    f = f_ref[...]
    wa = ew1_ref[0:DIM]
    wb = ew1_ref[DIM:2 * DIM]
    wd = ew1_ref[2 * DIM:2 * DIM + 1]
    ua = jnp.dot(f, wa, preferred_element_type=jnp.float32)
    fj_all = jnp.concatenate([fj_ref[0], fj_ref[1], fj_ref[2]],
                             axis=0)
    dk = jnp.concatenate([dsel_ref[0], dsel_ref[1], dsel_ref[2]],
                         axis=0)
    z = (jnp.concatenate([ua, ua, ua], axis=0)
         + jnp.dot(fj_all, wb, preferred_element_type=jnp.float32)
         + dk * wd
         + eb1_ref[...][None, :])
    m = _silu(z)
    mk = _silu(jnp.dot(m, ew2_ref[...], preferred_element_type=jnp.float32)
               + eb2_ref[...][None, :])
    mi = mk[0:BRL] + mk[BRL:2 * BRL] + mk[2 * BRL:3 * BRL]
    h = _silu(jnp.dot(f, nw1_ref[0:DIM], preferred_element_type=jnp.float32)
              + jnp.dot(mi, nw1_ref[DIM:DIM + MDIM],
                        preferred_element_type=jnp.float32)
              + nb1_ref[...][None, :])
    return f + jnp.dot(h, nw2_ref[...],
                       preferred_element_type=jnp.float32) \
        + nb2_ref[...][None, :]


def _layer_body(f_ref, fj_ref, dsel_ref, ew1_ref, eb1_ref, ew2_ref, eb2_ref,
                nw1_ref, nb1_ref, nw2_ref, nb2_ref, out_ref):
    out_ref[...] = _layer_compute(f_ref, fj_ref, dsel_ref, ew1_ref, eb1_ref,
                                  ew2_ref, eb2_ref, nw1_ref, nb1_ref,
                                  nw2_ref, nb2_ref)


def _layer_final_body(f_ref, fj_ref, dsel_ref, ew1_ref, eb1_ref, ew2_ref,
                      eb2_ref, nw1_ref, nb1_ref, nw2_ref, nb2_ref,
                      lw_ref, lb_ref, out_ref):
    fnew = _layer_compute(f_ref, fj_ref, dsel_ref, ew1_ref, eb1_ref,
                          ew2_ref, eb2_ref, nw1_ref, nb1_ref,
                          nw2_ref, nb2_ref)
    out_ref[...] = jnp.dot(fnew, lw_ref[...],
                           preferred_element_type=jnp.float32) \
        + lb_ref[...][None, :]


def _layer_specs():
    def full(shape):
        nzero = len(shape)
        return pl.BlockSpec(shape, lambda b, _n=nzero: (0,) * _n)
    return [
        pl.BlockSpec((BRL, DIM), lambda b: (b, 0)),
        pl.BlockSpec((K, BRL, DIM), lambda b: (0, b, 0)),
        pl.BlockSpec((K, BRL, 1), lambda b: (0, b, 0)),
        full((EIN, H1)),
        full((H1,)),
        full((H1, MDIM)),
        full((MDIM,)),
        full((DIM + MDIM, 2 * DIM)),
        full((2 * DIM,)),
        full((2 * DIM, DIM)),
        full((DIM,)),
    ]


def _layer(f, fj, dsel3, p):
    return pl.pallas_call(
        _layer_body,
        grid=(NBLKL,),
        in_specs=_layer_specs(),
        out_specs=pl.BlockSpec((BRL, DIM), lambda b: (b, 0)),
        out_shape=jax.ShapeDtypeStruct((N, DIM), jnp.float32),
    )(f, fj, dsel3, p['eW1'], p['eb1'], p['eW2'], p['eb2'],
      p['nW1'], p['nb1'], p['nW2'], p['nb2'])


def _layer_final(f, fj, dsel3, p, lw, lb):
    def full(shape):
        nzero = len(shape)
        return pl.BlockSpec(shape, lambda b, _n=nzero: (0,) * _n)
    return pl.pallas_call(
        _layer_final_body,
        grid=(NBLKL,),
        in_specs=_layer_specs() + [full((DIM, PWM)), full((PWM,))],
        out_specs=pl.BlockSpec((BRL, PWM), lambda b: (b, 0)),
        out_shape=jax.ShapeDtypeStruct((N, PWM), jnp.float32),
    )(f, fj, dsel3, p['eW1'], p['eb1'], p['eW2'], p['eb2'],
      p['nW1'], p['nb1'], p['nW2'], p['nb2'], lw, lb)


def kernel(feats, coords, params):
    f = feats[0]
    c = coords[0]
    cpad = jnp.pad(c, ((0, 0), (0, 5)))
    ct = cpad.T

    idx, dsel = _knn(cpad, ct)
    idx_flat = idx.reshape(-1)
    dsel3 = dsel[:, :, None]

    layers = params['layers']
    for p in layers[:-1]:
        fj = _gather_sc(f, idx_flat).reshape(K, N, DIM)
        f = _layer(f, fj, dsel3, p)
    fj = _gather_sc(f, idx_flat).reshape(K, N, DIM)
    out = _layer_final(f, fj, dsel3, layers[-1],
                       params['lin']['W'], params['lin']['b'])
    return out[None]


# speedup vs baseline: 1.3038x; 1.0418x over previous
import functools

import jax
import jax.numpy as jnp
from jax import lax
from jax.experimental import pallas as pl
from jax.experimental.pallas import tpu as pltpu
from jax.experimental.pallas import tpu_sc as plsc

N = 2048
DIM = 128
MDIM = 16
K = 3
PWM = 20
EIN = 2 * DIM + 1
H1 = 2 * EIN
BR = 512
NBLK = N // BR
BRL = 1024
NBLKL = N // BRL


def _silu(x):
    return x * (0.5 * jnp.tanh(0.5 * x) + 0.5)


def _knn_body(cpad_ref, ct_ref, idx_ref, dsel_ref):
    cr = cpad_ref[...]
    ct = ct_ref[...]
    acc = jnp.zeros((BR, N), jnp.float32)
    for d in range(3):
        diff = cr[:, d:d + 1] - ct[d:d + 1, :]
        acc = acc + diff * diff
    col = lax.broadcasted_iota(jnp.int32, (BR, N), 1)
    dist = acc
    idxs = []
    vals = []
    for t in range(K):
        m = jnp.min(dist, axis=1, keepdims=True)
        am = jnp.min(jnp.where(dist == m, col, N), axis=1,
                     keepdims=True)
        vals.append(m)
        idxs.append(am)
        if t + 1 < K:
            dist = jnp.where(col == am, jnp.inf, dist)
    idx_ref[...] = jnp.concatenate(idxs, axis=1).T
    dsel_ref[...] = jnp.concatenate(vals, axis=1).T


def _knn(cpad, ct):
    return pl.pallas_call(
        _knn_body,
        grid=(NBLK,),
        in_specs=[
            pl.BlockSpec((BR, 8), lambda b: (b, 0)),
            pl.BlockSpec((8, N), lambda b: (0, 0)),
        ],
        out_specs=[
            pl.BlockSpec((K, BR), lambda b: (0, b)),
            pl.BlockSpec((K, BR), lambda b: (0, b)),
        ],
        out_shape=[
            jax.ShapeDtypeStruct((K, N), jnp.int32),
            jax.ShapeDtypeStruct((K, N), jnp.float32),
        ],
    )(cpad, ct)


def _make_sc_gather():
    info = plsc.get_sparse_core_info()
    nc, ns = info.num_cores, info.num_subcores
    nw = nc * ns
    b_total = K * N
    b_per_w = b_total // nw
    mesh = plsc.VectorSubcoreMesh(core_axis_name="c", subcore_axis_name="s")

    @functools.partial(
        pl.kernel,
        mesh=mesh,
        out_type=jax.ShapeDtypeStruct((b_total, DIM), jnp.float32),
        scratch_types=[
            pltpu.VMEM((b_per_w,), jnp.int32),
            pltpu.VMEM((b_per_w, DIM), jnp.float32),
            pltpu.SemaphoreType.DMA,
        ],
    )
    def gather(table_hbm, idx_hbm, out_hbm, idx_v, rows_v, sem):
        wid = lax.axis_index("s") * nc + lax.axis_index("c")
        base = wid * b_per_w
        pltpu.sync_copy(idx_hbm.at[pl.ds(base, b_per_w)], idx_v)
        pltpu.async_copy(table_hbm.at[idx_v], rows_v, sem).wait()
        pltpu.sync_copy(rows_v, out_hbm.at[pl.ds(base, b_per_w)])

    return gather


_SC_GATHER_CACHE = []


def _gather_sc(table, idx):
    if not _SC_GATHER_CACHE:
        _SC_GATHER_CACHE.append(_make_sc_gather())
    return _SC_GATHER_CACHE[0](table, idx)


def _layer_compute(f_ref, fj_ref, dsel_ref, ew1_ref, eb1_ref, ew2_ref,
                   eb2_ref, nw1_ref, nb1_ref, nw2_ref, nb2_ref):
    f = f_ref[...]
    wa = ew1_ref[0:DIM]
    wb = ew1_ref[DIM:2 * DIM]
    wd = ew1_ref[2 * DIM:2 * DIM + 1]
    ua = (jnp.dot(f, wa, preferred_element_type=jnp.float32)
          + eb1_ref[...][None, :])
    mi = jnp.zeros((BRL, MDIM), jnp.float32)
    for k in range(K):
        z = (ua
             + jnp.dot(fj_ref[k], wb, preferred_element_type=jnp.float32)
             + dsel_ref[k] * wd)
        m = _silu(z)
        mi = mi + _silu(jnp.dot(m, ew2_ref[...],
                                preferred_element_type=jnp.float32)
                        + eb2_ref[...][None, :])
    h = _silu(jnp.dot(f, nw1_ref[0:DIM], preferred_element_type=jnp.float32)
              + jnp.dot(mi, nw1_ref[DIM:DIM + MDIM],
                        preferred_element_type=jnp.float32)
              + nb1_ref[...][None, :])
    return f + jnp.dot(h, nw2_ref[...],
                       preferred_element_type=jnp.float32) \
        + nb2_ref[...][None, :]


def _layer_body(f_ref, fj_ref, dsel_ref, ew1_ref, eb1_ref, ew2_ref, eb2_ref,
                nw1_ref, nb1_ref, nw2_ref, nb2_ref, out_ref):
    out_ref[...] = _layer_compute(f_ref, fj_ref, dsel_ref, ew1_ref, eb1_ref,
                                  ew2_ref, eb2_ref, nw1_ref, nb1_ref,
                                  nw2_ref, nb2_ref)


def _layer_final_body(f_ref, fj_ref, dsel_ref, ew1_ref, eb1_ref, ew2_ref,
                      eb2_ref, nw1_ref, nb1_ref, nw2_ref, nb2_ref,
                      lw_ref, lb_ref, out_ref):
    fnew = _layer_compute(f_ref, fj_ref, dsel_ref, ew1_ref, eb1_ref,
                          ew2_ref, eb2_ref, nw1_ref, nb1_ref,
                          nw2_ref, nb2_ref)
    out_ref[...] = jnp.dot(fnew, lw_ref[...],
                           preferred_element_type=jnp.float32) \
        + lb_ref[...][None, :]


def _layer_specs():
    def full(shape):
        nzero = len(shape)
        return pl.BlockSpec(shape, lambda b, _n=nzero: (0,) * _n)
    return [
        pl.BlockSpec((BRL, DIM), lambda b: (b, 0)),
        pl.BlockSpec((K, BRL, DIM), lambda b: (0, b, 0)),
        pl.BlockSpec((K, BRL, 1), lambda b: (0, b, 0)),
        full((EIN, H1)),
        full((H1,)),
        full((H1, MDIM)),
        full((MDIM,)),
        full((DIM + MDIM, 2 * DIM)),
        full((2 * DIM,)),
        full((2 * DIM, DIM)),
        full((DIM,)),
    ]


def _layer(f, fj, dsel3, p):
    return pl.pallas_call(
        _layer_body,
        grid=(NBLKL,),
        in_specs=_layer_specs(),
        out_specs=pl.BlockSpec((BRL, DIM), lambda b: (b, 0)),
        out_shape=jax.ShapeDtypeStruct((N, DIM), jnp.float32),
    )(f, fj, dsel3, p['eW1'], p['eb1'], p['eW2'], p['eb2'],
      p['nW1'], p['nb1'], p['nW2'], p['nb2'])


def _layer_final(f, fj, dsel3, p, lw, lb):
    def full(shape):
        nzero = len(shape)
        return pl.BlockSpec(shape, lambda b, _n=nzero: (0,) * _n)
    return pl.pallas_call(
        _layer_final_body,
        grid=(NBLKL,),
        in_specs=_layer_specs() + [full((DIM, PWM)), full((PWM,))],
        out_specs=pl.BlockSpec((BRL, PWM), lambda b: (b, 0)),
        out_shape=jax.ShapeDtypeStruct((N, PWM), jnp.float32),
    )(f, fj, dsel3, p['eW1'], p['eb1'], p['eW2'], p['eb2'],
      p['nW1'], p['nb1'], p['nW2'], p['nb2'], lw, lb)


def kernel(feats, coords, params):
    f = feats[0]
    c = coords[0]
    cpad = jnp.pad(c, ((0, 0), (0, 5)))
    ct = cpad.T

    idx, dsel = _knn(cpad, ct)
    idx_flat = idx.reshape(-1)
    dsel3 = dsel[:, :, None]

    layers = params['layers']
    for p in layers[:-1]:
        fj = _gather_sc(f, idx_flat).reshape(K, N, DIM)
        f = _layer(f, fj, dsel3, p)
    fj = _gather_sc(f, idx_flat).reshape(K, N, DIM)
    out = _layer_final(f, fj, dsel3, layers[-1],
                       params['lin']['W'], params['lin']['b'])
    return out[None]


# speedup vs baseline: 1.3122x; 1.0064x over previous
import functools

import jax
import jax.numpy as jnp
from jax import lax
from jax.experimental import pallas as pl
from jax.experimental.pallas import tpu as pltpu
from jax.experimental.pallas import tpu_sc as plsc

N = 2048
DIM = 128
MDIM = 16
K = 3
PWM = 20
EIN = 2 * DIM + 1
H1 = 2 * EIN
BR = 512
NBLK = N // BR
BRL = 1024
NBLKL = N // BRL


def _silu(x):
    return x * (0.5 * jnp.tanh(0.5 * x) + 0.5)


def _knn_body(c_ref, ct_ref, idx_ref, dsel_ref):
    cr = c_ref[...]
    ct = ct_ref[...]
    acc = None
    for d in range(3):
        diff = cr[:, d:d + 1] - ct[d:d + 1, :]
        sq = diff * diff
        acc = sq if acc is None else acc + sq
    col = lax.broadcasted_iota(jnp.int32, (BR, N), 1)
    dist = acc
    idxs = []
    vals = []
    for t in range(K):
        m = jnp.min(dist, axis=1, keepdims=True)
        am = jnp.min(jnp.where(dist == m, col, N), axis=1,
                     keepdims=True)
        vals.append(m)
        idxs.append(am)
        if t + 1 < K:
            dist = jnp.where(col == am, jnp.inf, dist)
    idx_ref[...] = jnp.concatenate(idxs, axis=1).T
    dsel_ref[...] = jnp.concatenate(vals, axis=1).T


def _knn(c, ct):
    return pl.pallas_call(
        _knn_body,
        grid=(NBLK,),
        in_specs=[
            pl.BlockSpec((BR, 3), lambda b: (b, 0)),
            pl.BlockSpec((3, N), lambda b: (0, 0)),
        ],
        out_specs=[
            pl.BlockSpec((K, BR), lambda b: (0, b)),
            pl.BlockSpec((K, BR), lambda b: (0, b)),
        ],
        out_shape=[
            jax.ShapeDtypeStruct((K, N), jnp.int32),
            jax.ShapeDtypeStruct((K, N), jnp.float32),
        ],
    )(c, ct)


def _make_sc_gather():
    info = plsc.get_sparse_core_info()
    nc, ns = info.num_cores, info.num_subcores
    nw = nc * ns
    b_total = K * N
    b_per_w = b_total // nw
    mesh = plsc.VectorSubcoreMesh(core_axis_name="c", subcore_axis_name="s")

    @functools.partial(
        pl.kernel,
        mesh=mesh,
        out_type=jax.ShapeDtypeStruct((b_total, DIM), jnp.float32),
        scratch_types=[
            pltpu.VMEM((b_per_w,), jnp.int32),
            pltpu.VMEM((b_per_w, DIM), jnp.float32),
            pltpu.SemaphoreType.DMA,
        ],
    )
    def gather(table_hbm, idx_hbm, out_hbm, idx_v, rows_v, sem):
        wid = lax.axis_index("s") * nc + lax.axis_index("c")
        base = wid * b_per_w
        pltpu.sync_copy(idx_hbm.at[pl.ds(base, b_per_w)], idx_v)
        pltpu.async_copy(table_hbm.at[idx_v], rows_v, sem).wait()
        pltpu.sync_copy(rows_v, out_hbm.at[pl.ds(base, b_per_w)])

    return gather


_SC_GATHER_CACHE = []


def _gather_sc(table, idx):
    if not _SC_GATHER_CACHE:
        _SC_GATHER_CACHE.append(_make_sc_gather())
    return _SC_GATHER_CACHE[0](table, idx)


def _layer_compute(f_ref, fj_ref, dsel_ref, ew1_ref, eb1_ref, ew2_ref,
                   eb2_ref, nw1_ref, nb1_ref, nw2_ref, nb2_ref):
    f = f_ref[...]
    wa = ew1_ref[0:DIM]
    wb = ew1_ref[DIM:2 * DIM]
    wd = ew1_ref[2 * DIM:2 * DIM + 1]
    ua = (jnp.dot(f, wa, preferred_element_type=jnp.float32)
          + eb1_ref[...][None, :])
    mi = jnp.zeros((BRL, MDIM), jnp.float32)
    for k in range(K):
        z = (ua
             + jnp.dot(fj_ref[k], wb, preferred_element_type=jnp.float32)
             + dsel_ref[k] * wd)
        m = _silu(z)
        mi = mi + _silu(jnp.dot(m, ew2_ref[...],
                                preferred_element_type=jnp.float32)
                        + eb2_ref[...][None, :])
    h = _silu(jnp.dot(f, nw1_ref[0:DIM], preferred_element_type=jnp.float32)
              + jnp.dot(mi, nw1_ref[DIM:DIM + MDIM],
                        preferred_element_type=jnp.float32)
              + nb1_ref[...][None, :])
    return f + jnp.dot(h, nw2_ref[...],
                       preferred_element_type=jnp.float32) \
        + nb2_ref[...][None, :]


def _layer_body(f_ref, fj_ref, dsel_ref, ew1_ref, eb1_ref, ew2_ref, eb2_ref,
                nw1_ref, nb1_ref, nw2_ref, nb2_ref, out_ref):
    out_ref[...] = _layer_compute(f_ref, fj_ref, dsel_ref, ew1_ref, eb1_ref,
                                  ew2_ref, eb2_ref, nw1_ref, nb1_ref,
                                  nw2_ref, nb2_ref)


def _layer_final_body(f_ref, fj_ref, dsel_ref, ew1_ref, eb1_ref, ew2_ref,
                      eb2_ref, nw1_ref, nb1_ref, nw2_ref, nb2_ref,
                      lw_ref, lb_ref, out_ref):
    fnew = _layer_compute(f_ref, fj_ref, dsel_ref, ew1_ref, eb1_ref,
                          ew2_ref, eb2_ref, nw1_ref, nb1_ref,
                          nw2_ref, nb2_ref)
    out_ref[...] = jnp.dot(fnew, lw_ref[...],
                           preferred_element_type=jnp.float32) \
        + lb_ref[...][None, :]


def _layer_specs():
    def full(shape):
        nzero = len(shape)
        return pl.BlockSpec(shape, lambda b, _n=nzero: (0,) * _n)
    return [
        pl.BlockSpec((BRL, DIM), lambda b: (b, 0)),
        pl.BlockSpec((K, BRL, DIM), lambda b: (0, b, 0)),
        pl.BlockSpec((K, BRL, 1), lambda b: (0, b, 0)),
        full((EIN, H1)),
        full((H1,)),
        full((H1, MDIM)),
        full((MDIM,)),
        full((DIM + MDIM, 2 * DIM)),
        full((2 * DIM,)),
        full((2 * DIM, DIM)),
        full((DIM,)),
    ]


def _layer(f, fj, dsel3, p):
    return pl.pallas_call(
        _layer_body,
        grid=(NBLKL,),
        in_specs=_layer_specs(),
        out_specs=pl.BlockSpec((BRL, DIM), lambda b: (b, 0)),
        out_shape=jax.ShapeDtypeStruct((N, DIM), jnp.float32),
    )(f, fj, dsel3, p['eW1'], p['eb1'], p['eW2'], p['eb2'],
      p['nW1'], p['nb1'], p['nW2'], p['nb2'])


def _layer_final(f, fj, dsel3, p, lw, lb):
    def full(shape):
        nzero = len(shape)
        return pl.BlockSpec(shape, lambda b, _n=nzero: (0,) * _n)
    return pl.pallas_call(
        _layer_final_body,
        grid=(NBLKL,),
        in_specs=_layer_specs() + [full((DIM, PWM)), full((PWM,))],
        out_specs=pl.BlockSpec((BRL, PWM), lambda b: (b, 0)),
        out_shape=jax.ShapeDtypeStruct((N, PWM), jnp.float32),
    )(f, fj, dsel3, p['eW1'], p['eb1'], p['eW2'], p['eb2'],
      p['nW1'], p['nb1'], p['nW2'], p['nb2'], lw, lb)


def kernel(feats, coords, params):
    f = feats[0]
    c = coords[0]
    ct = c.T

    idx, dsel = _knn(c, ct)
    idx_flat = idx.reshape(-1)
    dsel3 = dsel[:, :, None]

    layers = params['layers']
    for p in layers[:-1]:
        fj = _gather_sc(f, idx_flat).reshape(K, N, DIM)
        f = _layer(f, fj, dsel3, p)
    fj = _gather_sc(f, idx_flat).reshape(K, N, DIM)
    out = _layer_final(f, fj, dsel3, layers[-1],
                       params['lin']['W'], params['lin']['b'])
    return out[None]
